# baseline (speedup 1.0000x reference)
import functools

import jax
import jax.numpy as jnp
import numpy as np
from jax import lax
from jax.experimental import pallas as pl
from jax.experimental.pallas import tpu as pltpu
from jax.experimental.pallas import tpu_sc as plsc

_CUTOFF = 5.0
_NS = 32
_CH = 64
_NTILE = 16
_ZROWS = 80


def _silu(x):
    return x * jax.nn.sigmoid(x)


def _pack32(x):
    r = x.shape[0]
    return jnp.stack([x[:, 0:_NS // 2], x[:, _NS // 2:]], axis=-1).reshape(
        r, _NS)


def _node_body(ss_ref, sv_ref, w1_ref, b1_ref, w2_ref, b2_ref,
               ta_ref, t0_ref, t1_ref, t2_ref):
    x = ss_ref[...]
    h = jnp.dot(x, w1_ref[...], preferred_element_type=jnp.float32) + b1_ref[...]
    h = _silu(h)
    so = jnp.dot(h, w2_ref[...], preferred_element_type=jnp.float32) + b2_ref[...]
    so0 = so[:, 0:_NS]
    cpart = _pack32(so[:, _NS:2 * _NS])
    zpad = jnp.zeros_like(so0)
    bf16 = jnp.bfloat16
    ta_ref[...] = jnp.concatenate(
        [_pack32(x * so[:, 2 * _NS:]), zpad], axis=1).astype(bf16)
    v = sv_ref[...]
    t0_ref[...] = jnp.concatenate(
        [_pack32(v[:, 0, :] * so0), cpart], axis=1).astype(bf16)
    t1_ref[...] = jnp.concatenate(
        [_pack32(v[:, 1, :] * so0), cpart], axis=1).astype(bf16)
    t2_ref[...] = jnp.concatenate(
        [_pack32(v[:, 2, :] * so0), cpart], axis=1).astype(bf16)


def _edge_body(est_ref, edt_ref, evt_ref, fwt_ref, fb_ref,
               ed_out_ref, aux_ref, *, e_total, blk):
    i = pl.program_id(0)
    ft = jnp.dot(fwt_ref[...], est_ref[...],
                 preferred_element_type=jnp.float32) + fb_ref[...]
    col = i * blk + lax.broadcasted_iota(jnp.int32, (1, blk), 1)
    valid = col < e_total
    ft = jnp.where(valid, ft, 0.0)
    dist = edt_ref[...]
    cc = jnp.where(dist < _CUTOFF,
                   0.5 * (jnp.cos(np.pi * dist / _CUTOFF) + 1.0), 0.0)
    cc = jnp.where(valid, cc, 0.0)
    evt = evt_ref[...]
    nrm = jnp.sqrt(jnp.sum(evt * evt, axis=0, keepdims=True))
    evnt = evt / jnp.maximum(nrm, 1e-12)
    evnt = jnp.where(valid, evnt, 0.0)
    aux_ref[...] = jnp.concatenate(
        [evnt, cc, jnp.zeros((4, blk), jnp.float32)], axis=0)
    f = ft.T
    ed_out_ref[...] = jnp.concatenate(
        [_pack32(f[:, 0:_NS]), _pack32(f[:, _NS:2 * _NS]),
         _pack32(f[:, 2 * _NS:]), jnp.zeros((blk, _NS), jnp.float32)],
        axis=1).astype(jnp.bfloat16)


def _sc_body(src_ref, dst_ref, ta, t0, t1, t2, ed, aux,
             m_s, m_v0, m_v1, m_v2,
             zbuf,
             src_a, src_b, dst_a, dst_b,
             bufa_a, bufa_b, bufb_a, bufb_b,
             ev_a, ev_b, cc_a, cc_b, msg_a, msg_b,
             semd_a, semd_b, semsi_a, semsi_b,
             semdi_a, semdi_b, semsc_a, semsc_b,
             acc,
             *, n_nodes, e_pad):
    c = lax.axis_index("c")
    s = lax.axis_index("s")
    per_tile = e_pad // _NTILE
    chunks = per_tile // _CH
    n_rowchunks = n_nodes // _ZROWS
    max_rc_per_tile = (n_rowchunks + _NTILE - 1) // _NTILE

    @plsc.parallel_loop(0, _ZROWS)
    def _(r):
        z = jnp.zeros((16,), jnp.float32)
        zbuf[r, pl.ds(0, 16)] = z
        zbuf[r, pl.ds(16, 16)] = z

    def each_rowchunk(fn):
        def body(k, carry):
            j = s + k * _NTILE

            @pl.when(j < n_rowchunks)
            def _():
                fn(j)
            return carry
        lax.fori_loop(0, max_rc_per_tile, body, 0)

    def zero_acc():
        each_rowchunk(
            lambda j: pltpu.sync_copy(zbuf, acc.at[pl.ds(j * _ZROWS, _ZROWS)]))

    set_a = (src_a, dst_a, bufa_a, bufb_a, ev_a, cc_a, msg_a,
             semd_a, semsi_a, semdi_a, semsc_a)
    set_b = (src_b, dst_b, bufa_b, bufb_b, ev_b, cc_b, msg_b,
             semd_b, semsi_b, semdi_b, semsc_b)

    def e0_of(k):
        return s * per_tile + k * _CH

    last = chunks - 1

    def run_group(table, is_vec, dd):
        fw_col = 0 if is_vec else 2 * _NS

        def data_copies(k, S):
            (srci, dsti, bufa, bufb, bufev, bufcc, msgb,
             semd, semsi, semdi, semsc) = S
            e0 = e0_of(k)
            cps = [
                pltpu.make_async_copy(table.at[srci], bufa, semd),
                pltpu.make_async_copy(
                    ed.at[pl.ds(e0, _CH), pl.ds(fw_col, 2 * _NS)], bufb, semd),
                pltpu.make_async_copy(
                    aux.at[3, pl.ds(e0, _CH)], bufcc, semd),
            ]
            if is_vec:
                cps.append(pltpu.make_async_copy(
                    aux.at[dd, pl.ds(e0, _CH)], bufev, semd))
            return cps

        def issue_data(k, S):
            for cp in data_copies(k, S):
                cp.start()

        def drain_data(k, S):
            for cp in data_copies(k, S):
                cp.wait()

        def src_copy(k, S):
            return pltpu.make_async_copy(
                src_ref.at[pl.ds(e0_of(k), _CH)], S[0], S[8])

        def dst_copy(k, S):
            return pltpu.make_async_copy(
                dst_ref.at[pl.ds(e0_of(k), _CH)], S[1], S[9])

        def scatter_copy(S):
            return pltpu.make_async_copy(S[6], acc.at[S[1]], S[10])

        def scatter_start(S):
            scatter_copy(S).start(add=True)

        def compute(S):
            (srci, dsti, bufa, bufb, bufev, bufcc, msgb,
             semd, semsi, semdi, semsc) = S

            ilv = plsc.PackFormat.INTERLEAVED

            @plsc.parallel_loop(0, _CH, unroll=2)
            def _(e):
                esplat = jnp.full((16,), e, jnp.int32)
                x0, x1 = plsc.unpack(bufa[e, pl.ds(0, 2 * 16)], format=ilv)
                f0, f1 = plsc.unpack(bufb[e, pl.ds(0, 2 * 16)], format=ilv)
                cv = plsc.load_gather(bufcc, [esplat])
                if is_vec:
                    c0, c1 = plsc.unpack(
                        bufa[e, pl.ds(2 * 16, 2 * 16)], format=ilv)
                    f20, f21 = plsc.unpack(
                        bufb[e, pl.ds(2 * 16, 2 * 16)], format=ilv)
                    ev = plsc.load_gather(bufev, [esplat])
                    msgb[e, pl.ds(0, 16)] = (x0 * f0 + (c0 * f20) * ev) * cv
                    msgb[e, pl.ds(16, 16)] = (x1 * f1 + (c1 * f21) * ev) * cv
                else:
                    msgb[e, pl.ds(0, 16)] = (x0 * f0) * cv
                    msgb[e, pl.ds(16, 16)] = (x1 * f1) * cv

        def clamp(k):
            if isinstance(k, int):
                return min(k, last)
            return jnp.minimum(k, last)

        def iter_body(k, P, Q):
            scatter_copy(Q).wait()
            dst_copy(clamp(k + 1), Q).start()
            src_copy(clamp(k + 1), Q).wait()
            issue_data(clamp(k + 1), Q)
            drain_data(k, P)
            src_copy(clamp(k + 2), P).start()
            compute(P)
            dst_copy(k, P).wait()
            scatter_start(P)

        pltpu.sync_copy(src_ref.at[pl.ds(e0_of(0), _CH)], src_a)
        pltpu.sync_copy(dst_ref.at[pl.ds(e0_of(0), _CH)], dst_a)
        issue_data(0, set_a)
        src_copy(1, set_b).start()
        dst_copy(1, set_b).start()
        src_copy(1, set_b).wait()
        issue_data(1, set_b)
        drain_data(0, set_a)
        src_copy(2, set_a).start()
        compute(set_a)
        scatter_start(set_a)

        iter_body(1, set_b, set_a)

        def pair(kk, carry):
            k = 2 * kk
            iter_body(k, set_a, set_b)
            iter_body(k + 1, set_b, set_a)
            return carry
        lax.fori_loop(1, chunks // 2, pair, 0)

        scatter_copy(set_b).wait()
        drain_data(clamp(last + 1), set_a)
        src_copy(clamp(last + 2), set_b).wait()
        dst_copy(clamp(last + 1), set_a).wait()

    def run_scalar(table):
        run_group(table, False, 0)

    def run_vec(table, dd):
        run_group(table, True, dd)

    def flush(out_ref):
        each_rowchunk(
            lambda j: pltpu.sync_copy(acc.at[pl.ds(j * _ZROWS, _ZROWS)],
                                      out_ref.at[pl.ds(j * _ZROWS, _ZROWS)]))

    for gi in range(2):
        zero_acc()
        plsc.subcore_barrier()

        @pl.when(c == 0)
        def _():
            if gi == 0:
                run_scalar(ta)
            else:
                run_vec(t1, 1)

        @pl.when(c == 1)
        def _():
            if gi == 0:
                run_vec(t0, 0)
            else:
                run_vec(t2, 2)

        plsc.subcore_barrier()

        @pl.when(c == 0)
        def _():
            flush(m_s if gi == 0 else m_v1)

        @pl.when(c == 1)
        def _():
            flush(m_v0 if gi == 0 else m_v2)

        if gi == 0:
            plsc.subcore_barrier()


def _final_body(ms_ref, m0_ref, m1_ref, m2_ref, rs_ref, rv_ref,
                gw1_ref, gb1_ref, gw2_ref, gb2_ref, os_ref, ov_ref):
    ms = ms_ref[...]
    g = _silu(jnp.dot(ms, gw1_ref[...],
                      preferred_element_type=jnp.float32) + gb1_ref[...])
    g = jax.nn.sigmoid(jnp.dot(g, gw2_ref[...],
                               preferred_element_type=jnp.float32) + gb2_ref[...])
    ug_s = g[:, :_NS]
    ug_v = g[:, _NS:]
    os_ref[...] = ug_s * rs_ref[...] + (1.0 - ug_s) * ms
    mv = jnp.stack([m0_ref[...], m1_ref[...], m2_ref[...]], axis=1)
    ov_ref[...] = ug_v[:, None, :] * rv_ref[...] + (1.0 - ug_v[:, None, :]) * mv


def kernel(sender_node_state_scalar, sender_node_state_vector,
           receiver_node_state_scalar, receiver_node_state_vector,
           edge_state, edge_vector, edge_distance, edges,
           filter_W, filter_b, msg_W1, msg_b1, msg_W2, msg_b2,
           gate_W1, gate_b1, gate_W2, gate_b2):
    N, ns = sender_node_state_scalar.shape
    E = edges.shape[0]
    es_dim = edge_state.shape[1]
    assert ns == _NS
    assert N % _ZROWS == 0

    blk_e = _NTILE * _CH
    e_pad = ((E + blk_e - 1) // blk_e) * blk_e
    grid_e = e_pad // blk_e

    src = edges[:, 0].astype(jnp.int32)
    dst = edges[:, 1].astype(jnp.int32)
    pad = e_pad - E
    if pad:
        zpad = jnp.zeros((pad,), jnp.int32)
        src = jnp.concatenate([src, zpad])
        dst = jnp.concatenate([dst, zpad])

    f32 = jnp.float32

    Rn = 1000
    grid_n = N // Rn
    ta, t0, t1, t2 = pl.pallas_call(
        _node_body,
        grid=(grid_n,),
        in_specs=[
            pl.BlockSpec((Rn, ns), lambda i: (i, 0)),
            pl.BlockSpec((Rn, 3, ns), lambda i: (i, 0, 0)),
            pl.BlockSpec((ns, ns), lambda i: (0, 0)),
            pl.BlockSpec((1, ns), lambda i: (0, 0)),
            pl.BlockSpec((ns, 3 * ns), lambda i: (0, 0)),
            pl.BlockSpec((1, 3 * ns), lambda i: (0, 0)),
        ],
        out_specs=[
            pl.BlockSpec((Rn, 2 * ns), lambda i: (i, 0)),
            pl.BlockSpec((Rn, 2 * ns), lambda i: (i, 0)),
            pl.BlockSpec((Rn, 2 * ns), lambda i: (i, 0)),
            pl.BlockSpec((Rn, 2 * ns), lambda i: (i, 0)),
        ],
        out_shape=[
            jax.ShapeDtypeStruct((N, 2 * ns), jnp.bfloat16),
            jax.ShapeDtypeStruct((N, 2 * ns), jnp.bfloat16),
            jax.ShapeDtypeStruct((N, 2 * ns), jnp.bfloat16),
            jax.ShapeDtypeStruct((N, 2 * ns), jnp.bfloat16),
        ],
    )(sender_node_state_scalar, sender_node_state_vector,
      msg_W1, msg_b1.reshape(1, ns), msg_W2, msg_b2.reshape(1, 3 * ns))

    ed_a, aux_a = pl.pallas_call(
        functools.partial(_edge_body, e_total=E, blk=blk_e),
        grid=(grid_e,),
        in_specs=[
            pl.BlockSpec((es_dim, blk_e), lambda i: (0, i)),
            pl.BlockSpec((1, blk_e), lambda i: (0, i)),
            pl.BlockSpec((3, blk_e), lambda i: (0, i)),
            pl.BlockSpec((3 * ns, es_dim), lambda i: (0, 0)),
            pl.BlockSpec((3 * ns, 1), lambda i: (0, 0)),
        ],
        out_specs=[
            pl.BlockSpec((blk_e, 4 * ns), lambda i: (i, 0)),
            pl.BlockSpec((8, blk_e), lambda i: (0, i)),
        ],
        out_shape=[
            jax.ShapeDtypeStruct((e_pad, 4 * ns), jnp.bfloat16),
            jax.ShapeDtypeStruct((8, e_pad), f32),
        ],
    )(edge_state.T, edge_distance.T, edge_vector.T,
      filter_W.T, filter_b.reshape(3 * ns, 1))

    mesh = plsc.VectorSubcoreMesh(core_axis_name="c", subcore_axis_name="s",
                                  num_cores=2, num_subcores=_NTILE)
    m_s, m_v0, m_v1, m_v2 = pl.kernel(
        functools.partial(_sc_body, n_nodes=N, e_pad=e_pad),
        out_type=[jax.ShapeDtypeStruct((N, ns), f32)] * 4,
        mesh=mesh,
        compiler_params=pltpu.CompilerParams(needs_layout_passes=False,
                                             use_tc_tiling_on_sc=False),
        scratch_types=(
            [pltpu.VMEM((_ZROWS, ns), f32)]
            + [pltpu.VMEM((_CH,), jnp.int32)] * 4
            + [pltpu.VMEM((_CH, 2 * ns), jnp.bfloat16)] * 4
            + [pltpu.VMEM((_CH,), f32)] * 4
            + [pltpu.VMEM((_CH, ns), f32)] * 2
            + [pltpu.SemaphoreType.DMA] * 8
            + [pltpu.VMEM_SHARED((N, ns), f32)]
        ),
    )(src, dst, ta, t0, t1, t2, ed_a, aux_a)

    new_s, new_v = pl.pallas_call(
        _final_body,
        grid=(grid_n,),
        in_specs=[
            pl.BlockSpec((Rn, ns), lambda i: (i, 0)),
            pl.BlockSpec((Rn, ns), lambda i: (i, 0)),
            pl.BlockSpec((Rn, ns), lambda i: (i, 0)),
            pl.BlockSpec((Rn, ns), lambda i: (i, 0)),
            pl.BlockSpec((Rn, ns), lambda i: (i, 0)),
            pl.BlockSpec((Rn, 3, ns), lambda i: (i, 0, 0)),
            pl.BlockSpec((ns, 2 * ns), lambda i: (0, 0)),
            pl.BlockSpec((1, 2 * ns), lambda i: (0, 0)),
            pl.BlockSpec((2 * ns, 2 * ns), lambda i: (0, 0)),
            pl.BlockSpec((1, 2 * ns), lambda i: (0, 0)),
        ],
        out_specs=[
            pl.BlockSpec((Rn, ns), lambda i: (i, 0)),
            pl.BlockSpec((Rn, 3, ns), lambda i: (i, 0, 0)),
        ],
        out_shape=[
            jax.ShapeDtypeStruct((N, ns), f32),
            jax.ShapeDtypeStruct((N, 3, ns), f32),
        ],
    )(m_s, m_v0, m_v1, m_v2,
      receiver_node_state_scalar, receiver_node_state_vector,
      gate_W1, gate_b1.reshape(1, 2 * ns), gate_W2, gate_b2.reshape(1, 2 * ns))

    return new_s, new_v


# speedup vs baseline: 5.7947x; 5.7947x over previous
import functools

import jax
import jax.numpy as jnp
import numpy as np
from jax import lax
from jax.experimental import pallas as pl
from jax.experimental.pallas import tpu as pltpu
from jax.experimental.pallas import tpu_sc as plsc

_CUTOFF = 5.0
_NS = 32
_CH = 64
_NTILE = 16
_ZROWS = 80


def _silu(x):
    return x * jax.nn.sigmoid(x)


def _node_body(ss_ref, sv_ref, w1_ref, b1_ref, w2_ref, b2_ref,
               ta_ref, t0_ref, t1_ref, t2_ref):
    x = ss_ref[...]
    h = jnp.dot(x, w1_ref[...], preferred_element_type=jnp.float32) + b1_ref[...]
    h = _silu(h)
    so = jnp.dot(h, w2_ref[...], preferred_element_type=jnp.float32) + b2_ref[...]
    so0 = so[:, 0:_NS]
    cpart = so[:, _NS:2 * _NS]
    zpad = jnp.zeros_like(so0)
    bf16 = jnp.bfloat16
    ta_ref[...] = jnp.concatenate(
        [x * so[:, 2 * _NS:], zpad], axis=1).astype(bf16)
    v = sv_ref[...]
    t0_ref[...] = jnp.concatenate(
        [v[:, 0, :] * so0, cpart], axis=1).astype(bf16)
    t1_ref[...] = jnp.concatenate(
        [v[:, 1, :] * so0, cpart], axis=1).astype(bf16)
    t2_ref[...] = jnp.concatenate(
        [v[:, 2, :] * so0, cpart], axis=1).astype(bf16)


def _edge_body(est_ref, edt_ref, evt_ref, fwt_ref, fb_ref,
               ed_out_ref, aux_ref, *, e_total, blk):
    i = pl.program_id(0)
    ft = jnp.dot(fwt_ref[...], est_ref[...],
                 preferred_element_type=jnp.float32) + fb_ref[...]
    col = i * blk + lax.broadcasted_iota(jnp.int32, (1, blk), 1)
    valid = col < e_total
    ft = jnp.where(valid, ft, 0.0)
    dist = edt_ref[...]
    cc = jnp.where(dist < _CUTOFF,
                   0.5 * (jnp.cos(np.pi * dist / _CUTOFF) + 1.0), 0.0)
    cc = jnp.where(valid, cc, 0.0)
    evt = evt_ref[...]
    nrm = jnp.sqrt(jnp.sum(evt * evt, axis=0, keepdims=True))
    evnt = evt / jnp.maximum(nrm, 1e-12)
    evnt = jnp.where(valid, evnt, 0.0)
    aux_ref[...] = jnp.concatenate(
        [evnt, cc, jnp.zeros((4, blk), jnp.float32)], axis=0)
    f = ft.T
    ed_out_ref[...] = jnp.concatenate(
        [f, jnp.zeros((blk, _NS), jnp.float32)], axis=1).astype(jnp.bfloat16)


def _sc_body(src_ref, dst_ref, ta, t0, t1, t2, ed, aux,
             m_s, m_v0, m_v1, m_v2,
             zbuf,
             src_a, src_b, dst_a, dst_b,
             bufa_a, bufa_b, bufb_a, bufb_b,
             ev_a, ev_b, cc_a, cc_b, msg_a, msg_b,
             semd_a, semd_b, semsi_a, semsi_b,
             semdi_a, semdi_b, semsc_a, semsc_b,
             acc,
             *, n_nodes, e_pad):
    c = lax.axis_index("c")
    s = lax.axis_index("s")
    per_tile = e_pad // _NTILE
    chunks = per_tile // _CH
    n_rowchunks = n_nodes // _ZROWS
    max_rc_per_tile = (n_rowchunks + _NTILE - 1) // _NTILE

    @plsc.parallel_loop(0, _ZROWS)
    def _(r):
        z = jnp.zeros((16,), jnp.float32)
        zbuf[r, pl.ds(0, 16)] = z
        zbuf[r, pl.ds(16, 16)] = z

    def each_rowchunk(fn):
        def body(k, carry):
            j = s + k * _NTILE

            @pl.when(j < n_rowchunks)
            def _():
                fn(j)
            return carry
        lax.fori_loop(0, max_rc_per_tile, body, 0)

    def zero_acc():
        each_rowchunk(
            lambda j: pltpu.sync_copy(zbuf, acc.at[pl.ds(j * _ZROWS, _ZROWS)]))

    set_a = (src_a, dst_a, bufa_a, bufb_a, ev_a, cc_a, msg_a,
             semd_a, semsi_a, semdi_a, semsc_a)
    set_b = (src_b, dst_b, bufa_b, bufb_b, ev_b, cc_b, msg_b,
             semd_b, semsi_b, semdi_b, semsc_b)

    def e0_of(k):
        return s * per_tile + k * _CH

    last = chunks - 1

    def run_group(table, is_vec, dd):
        fw_col = 0 if is_vec else 2 * _NS

        def data_copies(k, S):
            (srci, dsti, bufa, bufb, bufev, bufcc, msgb,
             semd, semsi, semdi, semsc) = S
            e0 = e0_of(k)
            cps = [
                pltpu.make_async_copy(table.at[srci], bufa, semd),
                pltpu.make_async_copy(
                    ed.at[pl.ds(e0, _CH), pl.ds(fw_col, 2 * _NS)], bufb, semd),
                pltpu.make_async_copy(
                    aux.at[3, pl.ds(e0, _CH)], bufcc, semd),
            ]
            if is_vec:
                cps.append(pltpu.make_async_copy(
                    aux.at[dd, pl.ds(e0, _CH)], bufev, semd))
            return cps

        def issue_data(k, S):
            for cp in data_copies(k, S):
                cp.start()

        def drain_data(k, S):
            for cp in data_copies(k, S):
                cp.wait()

        def src_copy(k, S):
            return pltpu.make_async_copy(
                src_ref.at[pl.ds(e0_of(k), _CH)], S[0], S[8])

        def dst_copy(k, S):
            return pltpu.make_async_copy(
                dst_ref.at[pl.ds(e0_of(k), _CH)], S[1], S[9])

        def scatter_copy(S):
            return pltpu.make_async_copy(S[6], acc.at[S[1]], S[10])

        def scatter_start(S):
            scatter_copy(S).start(add=True)

        def compute(S):
            (srci, dsti, bufa, bufb, bufev, bufcc, msgb,
             semd, semsi, semdi, semsc) = S
            ilv = plsc.PackFormat.INTERLEAVED
            col_e = lax.iota(jnp.int32, 16) * 2
            col_o = col_e + 1

            @plsc.parallel_loop(0, _CH, unroll=2)
            def _(e):
                esplat = jnp.full((16,), e, jnp.int32)
                xe, xo = plsc.unpack(bufa[e, pl.ds(0, 32)], format=ilv)
                fe, fo = plsc.unpack(bufb[e, pl.ds(0, 32)], format=ilv)
                cv = plsc.load_gather(bufcc, [esplat])
                if is_vec:
                    ce, co = plsc.unpack(bufa[e, pl.ds(32, 32)], format=ilv)
                    f2e, f2o = plsc.unpack(bufb[e, pl.ds(32, 32)], format=ilv)
                    ev = plsc.load_gather(bufev, [esplat])
                    me = (xe * fe + (ce * f2e) * ev) * cv
                    mo = (xo * fo + (co * f2o) * ev) * cv
                else:
                    me = (xe * fe) * cv
                    mo = (xo * fo) * cv
                plsc.store_scatter(msgb, [esplat, col_e], me)
                plsc.store_scatter(msgb, [esplat, col_o], mo)

        def clamp(k):
            if isinstance(k, int):
                return min(k, last)
            return jnp.minimum(k, last)

        def iter_body(k, P, Q):
            scatter_copy(Q).wait()
            dst_copy(clamp(k + 1), Q).start()
            src_copy(clamp(k + 1), Q).wait()
            issue_data(clamp(k + 1), Q)
            drain_data(k, P)
            src_copy(clamp(k + 2), P).start()
            compute(P)
            dst_copy(k, P).wait()
            scatter_start(P)

        pltpu.sync_copy(src_ref.at[pl.ds(e0_of(0), _CH)], src_a)
        pltpu.sync_copy(dst_ref.at[pl.ds(e0_of(0), _CH)], dst_a)
        issue_data(0, set_a)
        src_copy(1, set_b).start()
        dst_copy(1, set_b).start()
        src_copy(1, set_b).wait()
        issue_data(1, set_b)
        drain_data(0, set_a)
        src_copy(2, set_a).start()
        compute(set_a)
        scatter_start(set_a)

        iter_body(1, set_b, set_a)

        def pair(kk, carry):
            k = 2 * kk
            iter_body(k, set_a, set_b)
            iter_body(k + 1, set_b, set_a)
            return carry
        lax.fori_loop(1, chunks // 2, pair, 0)

        scatter_copy(set_b).wait()
        drain_data(clamp(last + 1), set_a)
        src_copy(clamp(last + 2), set_b).wait()
        dst_copy(clamp(last + 1), set_a).wait()

    def run_scalar(table):
        run_group(table, False, 0)

    def run_vec(table, dd):
        run_group(table, True, dd)

    def flush(out_ref):
        each_rowchunk(
            lambda j: pltpu.sync_copy(acc.at[pl.ds(j * _ZROWS, _ZROWS)],
                                      out_ref.at[pl.ds(j * _ZROWS, _ZROWS)]))

    for gi in range(2):
        zero_acc()
        plsc.subcore_barrier()

        @pl.when(c == 0)
        def _():
            if gi == 0:
                run_scalar(ta)
            else:
                run_vec(t1, 1)

        @pl.when(c == 1)
        def _():
            if gi == 0:
                run_vec(t0, 0)
            else:
                run_vec(t2, 2)

        plsc.subcore_barrier()

        @pl.when(c == 0)
        def _():
            flush(m_s if gi == 0 else m_v1)

        @pl.when(c == 1)
        def _():
            flush(m_v0 if gi == 0 else m_v2)

        if gi == 0:
            plsc.subcore_barrier()


def _final_body(ms_ref, m0_ref, m1_ref, m2_ref, rs_ref, rv_ref,
                gw1_ref, gb1_ref, gw2_ref, gb2_ref, os_ref, ov_ref):
    ms = ms_ref[...]
    g = _silu(jnp.dot(ms, gw1_ref[...],
                      preferred_element_type=jnp.float32) + gb1_ref[...])
    g = jax.nn.sigmoid(jnp.dot(g, gw2_ref[...],
                               preferred_element_type=jnp.float32) + gb2_ref[...])
    ug_s = g[:, :_NS]
    ug_v = g[:, _NS:]
    os_ref[...] = ug_s * rs_ref[...] + (1.0 - ug_s) * ms
    mv = jnp.stack([m0_ref[...], m1_ref[...], m2_ref[...]], axis=1)
    ov_ref[...] = ug_v[:, None, :] * rv_ref[...] + (1.0 - ug_v[:, None, :]) * mv


def kernel(sender_node_state_scalar, sender_node_state_vector,
           receiver_node_state_scalar, receiver_node_state_vector,
           edge_state, edge_vector, edge_distance, edges,
           filter_W, filter_b, msg_W1, msg_b1, msg_W2, msg_b2,
           gate_W1, gate_b1, gate_W2, gate_b2):
    N, ns = sender_node_state_scalar.shape
    E = edges.shape[0]
    es_dim = edge_state.shape[1]
    assert ns == _NS
    assert N % _ZROWS == 0

    blk_e = _NTILE * _CH
    e_pad = ((E + blk_e - 1) // blk_e) * blk_e
    grid_e = e_pad // blk_e

    src = edges[:, 0].astype(jnp.int32)
    dst = edges[:, 1].astype(jnp.int32)
    pad = e_pad - E
    if pad:
        zpad = jnp.zeros((pad,), jnp.int32)
        src = jnp.concatenate([src, zpad])
        dst = jnp.concatenate([dst, zpad])

    f32 = jnp.float32

    Rn = 2000
    grid_n = N // Rn
    ta, t0, t1, t2 = pl.pallas_call(
        _node_body,
        grid=(grid_n,),
        in_specs=[
            pl.BlockSpec((Rn, ns), lambda i: (i, 0)),
            pl.BlockSpec((Rn, 3, ns), lambda i: (i, 0, 0)),
            pl.BlockSpec((ns, ns), lambda i: (0, 0)),
            pl.BlockSpec((1, ns), lambda i: (0, 0)),
            pl.BlockSpec((ns, 3 * ns), lambda i: (0, 0)),
            pl.BlockSpec((1, 3 * ns), lambda i: (0, 0)),
        ],
        out_specs=[
            pl.BlockSpec((Rn, 2 * ns), lambda i: (i, 0)),
            pl.BlockSpec((Rn, 2 * ns), lambda i: (i, 0)),
            pl.BlockSpec((Rn, 2 * ns), lambda i: (i, 0)),
            pl.BlockSpec((Rn, 2 * ns), lambda i: (i, 0)),
        ],
        out_shape=[
            jax.ShapeDtypeStruct((N, 2 * ns), jnp.bfloat16),
            jax.ShapeDtypeStruct((N, 2 * ns), jnp.bfloat16),
            jax.ShapeDtypeStruct((N, 2 * ns), jnp.bfloat16),
            jax.ShapeDtypeStruct((N, 2 * ns), jnp.bfloat16),
        ],
    )(sender_node_state_scalar, sender_node_state_vector,
      msg_W1, msg_b1.reshape(1, ns), msg_W2, msg_b2.reshape(1, 3 * ns))

    ed_a, aux_a = pl.pallas_call(
        functools.partial(_edge_body, e_total=E, blk=blk_e),
        grid=(grid_e,),
        in_specs=[
            pl.BlockSpec((es_dim, blk_e), lambda i: (0, i)),
            pl.BlockSpec((1, blk_e), lambda i: (0, i)),
            pl.BlockSpec((3, blk_e), lambda i: (0, i)),
            pl.BlockSpec((3 * ns, es_dim), lambda i: (0, 0)),
            pl.BlockSpec((3 * ns, 1), lambda i: (0, 0)),
        ],
        out_specs=[
            pl.BlockSpec((blk_e, 4 * ns), lambda i: (i, 0)),
            pl.BlockSpec((8, blk_e), lambda i: (0, i)),
        ],
        out_shape=[
            jax.ShapeDtypeStruct((e_pad, 4 * ns), jnp.bfloat16),
            jax.ShapeDtypeStruct((8, e_pad), f32),
        ],
    )(edge_state.T, edge_distance.T, edge_vector.T,
      filter_W.T, filter_b.reshape(3 * ns, 1))

    mesh = plsc.VectorSubcoreMesh(core_axis_name="c", subcore_axis_name="s",
                                  num_cores=2, num_subcores=_NTILE)
    m_s, m_v0, m_v1, m_v2 = pl.kernel(
        functools.partial(_sc_body, n_nodes=N, e_pad=e_pad),
        out_type=[jax.ShapeDtypeStruct((N, ns), f32)] * 4,
        mesh=mesh,
        compiler_params=pltpu.CompilerParams(needs_layout_passes=False,
                                             use_tc_tiling_on_sc=False),
        scratch_types=(
            [pltpu.VMEM((_ZROWS, ns), f32)]
            + [pltpu.VMEM((_CH,), jnp.int32)] * 4
            + [pltpu.VMEM((_CH, 2 * ns), jnp.bfloat16)] * 4
            + [pltpu.VMEM((_CH,), f32)] * 4
            + [pltpu.VMEM((_CH, ns), f32)] * 2
            + [pltpu.SemaphoreType.DMA] * 8
            + [pltpu.VMEM_SHARED((N, ns), f32)]
        ),
    )(src, dst, ta, t0, t1, t2, ed_a, aux_a)

    new_s, new_v = pl.pallas_call(
        _final_body,
        grid=(grid_n,),
        in_specs=[
            pl.BlockSpec((Rn, ns), lambda i: (i, 0)),
            pl.BlockSpec((Rn, ns), lambda i: (i, 0)),
            pl.BlockSpec((Rn, ns), lambda i: (i, 0)),
            pl.BlockSpec((Rn, ns), lambda i: (i, 0)),
            pl.BlockSpec((Rn, ns), lambda i: (i, 0)),
            pl.BlockSpec((Rn, 3, ns), lambda i: (i, 0, 0)),
            pl.BlockSpec((ns, 2 * ns), lambda i: (0, 0)),
            pl.BlockSpec((1, 2 * ns), lambda i: (0, 0)),
            pl.BlockSpec((2 * ns, 2 * ns), lambda i: (0, 0)),
            pl.BlockSpec((1, 2 * ns), lambda i: (0, 0)),
        ],
        out_specs=[
            pl.BlockSpec((Rn, ns), lambda i: (i, 0)),
            pl.BlockSpec((Rn, 3, ns), lambda i: (i, 0, 0)),
        ],
        out_shape=[
            jax.ShapeDtypeStruct((N, ns), f32),
            jax.ShapeDtypeStruct((N, 3, ns), f32),
        ],
    )(m_s, m_v0, m_v1, m_v2,
      receiver_node_state_scalar, receiver_node_state_vector,
      gate_W1, gate_b1.reshape(1, 2 * ns), gate_W2, gate_b2.reshape(1, 2 * ns))

    return new_s, new_v


# speedup vs baseline: 6.7272x; 1.1609x over previous
import functools

import jax
import jax.numpy as jnp
import numpy as np
from jax import lax
from jax.experimental import pallas as pl
from jax.experimental.pallas import tpu as pltpu
from jax.experimental.pallas import tpu_sc as plsc

_CUTOFF = 5.0
_NS = 32
_CH = 64
_NTILE = 16
_ZROWS = 80


def _silu(x):
    return x * jax.nn.sigmoid(x)


def _node_body(ss_ref, sv_ref, w1_ref, b1_ref, w2_ref, b2_ref,
               ta_ref, t0_ref, t1_ref, t2_ref):
    x = ss_ref[...]
    h = jnp.dot(x, w1_ref[...], preferred_element_type=jnp.float32) + b1_ref[...]
    h = _silu(h)
    so = jnp.dot(h, w2_ref[...], preferred_element_type=jnp.float32) + b2_ref[...]
    so0 = so[:, 0:_NS]
    cpart = so[:, _NS:2 * _NS]
    zpad = jnp.zeros_like(so0)
    ta_ref[...] = jnp.concatenate([x * so[:, 2 * _NS:], zpad], axis=1)
    v = sv_ref[...]
    t0_ref[...] = jnp.concatenate([v[:, 0, :] * so0, cpart], axis=1)
    t1_ref[...] = jnp.concatenate([v[:, 1, :] * so0, cpart], axis=1)
    t2_ref[...] = jnp.concatenate([v[:, 2, :] * so0, cpart], axis=1)


def _edge_body(est_ref, edt_ref, evt_ref, fwt_ref, fb_ref,
               ed_out_ref, aux_ref, *, e_total, blk):
    i = pl.program_id(0)
    ft = jnp.dot(fwt_ref[...], est_ref[...],
                 preferred_element_type=jnp.float32) + fb_ref[...]
    col = i * blk + lax.broadcasted_iota(jnp.int32, (1, blk), 1)
    valid = col < e_total
    ft = jnp.where(valid, ft, 0.0)
    dist = edt_ref[...]
    cc = jnp.where(dist < _CUTOFF,
                   0.5 * (jnp.cos(np.pi * dist / _CUTOFF) + 1.0), 0.0)
    cc = jnp.where(valid, cc, 0.0)
    evt = evt_ref[...]
    nrm = jnp.sqrt(jnp.sum(evt * evt, axis=0, keepdims=True))
    evnt = evt / jnp.maximum(nrm, 1e-12)
    evnt = jnp.where(valid, evnt, 0.0)
    aux_ref[...] = jnp.concatenate(
        [evnt, cc, jnp.zeros((4, blk), jnp.float32)], axis=0)
    f = ft.T
    ed_out_ref[...] = jnp.concatenate(
        [f, jnp.zeros((blk, _NS), jnp.float32)], axis=1)


def _sc_body(src_ref, dst_ref, ta, t0, t1, t2, ed, aux,
             m_s, m_v0, m_v1, m_v2,
             zbuf,
             src_a, src_b, dst_a, dst_b,
             bufa_a, bufa_b, bufb_a, bufb_b,
             ev_a, ev_b, cc_a, cc_b, msg_a, msg_b,
             semd_a, semd_b, semsi_a, semsi_b,
             semdi_a, semdi_b, semsc_a, semsc_b,
             acc,
             *, n_nodes, e_pad):
    c = lax.axis_index("c")
    s = lax.axis_index("s")
    per_tile = e_pad // _NTILE
    chunks = per_tile // _CH
    n_rowchunks = n_nodes // _ZROWS
    max_rc_per_tile = (n_rowchunks + _NTILE - 1) // _NTILE

    @plsc.parallel_loop(0, _ZROWS)
    def _(r):
        z = jnp.zeros((16,), jnp.float32)
        zbuf[r, pl.ds(0, 16)] = z
        zbuf[r, pl.ds(16, 16)] = z

    def each_rowchunk(fn):
        def body(k, carry):
            j = s + k * _NTILE

            @pl.when(j < n_rowchunks)
            def _():
                fn(j)
            return carry
        lax.fori_loop(0, max_rc_per_tile, body, 0)

    def zero_acc():
        each_rowchunk(
            lambda j: pltpu.sync_copy(zbuf, acc.at[pl.ds(j * _ZROWS, _ZROWS)]))

    set_a = (src_a, dst_a, bufa_a, bufb_a, ev_a, cc_a, msg_a,
             semd_a, semsi_a, semdi_a, semsc_a)
    set_b = (src_b, dst_b, bufa_b, bufb_b, ev_b, cc_b, msg_b,
             semd_b, semsi_b, semdi_b, semsc_b)

    def e0_of(k):
        return s * per_tile + k * _CH

    last = chunks - 1

    def run_group(table, is_vec, dd):
        fw_col = 0 if is_vec else 2 * _NS

        def data_copies(k, S):
            (srci, dsti, bufa, bufb, bufev, bufcc, msgb,
             semd, semsi, semdi, semsc) = S
            e0 = e0_of(k)
            cps = [
                pltpu.make_async_copy(table.at[srci], bufa, semd),
                pltpu.make_async_copy(
                    ed.at[pl.ds(e0, _CH), pl.ds(fw_col, 2 * _NS)], bufb, semd),
                pltpu.make_async_copy(
                    aux.at[3, pl.ds(e0, _CH)], bufcc, semd),
            ]
            if is_vec:
                cps.append(pltpu.make_async_copy(
                    aux.at[dd, pl.ds(e0, _CH)], bufev, semd))
            return cps

        def issue_data(k, S):
            for cp in data_copies(k, S):
                cp.start()

        def drain_data(k, S):
            for cp in data_copies(k, S):
                cp.wait()

        def src_copy(k, S):
            return pltpu.make_async_copy(
                src_ref.at[pl.ds(e0_of(k), _CH)], S[0], S[8])

        def dst_copy(k, S):
            return pltpu.make_async_copy(
                dst_ref.at[pl.ds(e0_of(k), _CH)], S[1], S[9])

        def scatter_copy(S):
            return pltpu.make_async_copy(S[6], acc.at[S[1]], S[10])

        def scatter_start(S):
            scatter_copy(S).start(add=True)

        def compute(S):
            (srci, dsti, bufa, bufb, bufev, bufcc, msgb,
             semd, semsi, semdi, semsc) = S

            @plsc.parallel_loop(0, _CH, unroll=2)
            def _(e):
                esplat = jnp.full((16,), e, jnp.int32)
                x0 = bufa[e, pl.ds(0, 16)]
                x1 = bufa[e, pl.ds(16, 16)]
                f0 = bufb[e, pl.ds(0, 16)]
                f1 = bufb[e, pl.ds(16, 16)]
                cv = plsc.load_gather(bufcc, [esplat])
                if is_vec:
                    c0 = bufa[e, pl.ds(32, 16)]
                    c1 = bufa[e, pl.ds(48, 16)]
                    f20 = bufb[e, pl.ds(32, 16)]
                    f21 = bufb[e, pl.ds(48, 16)]
                    ev = plsc.load_gather(bufev, [esplat])
                    msgb[e, pl.ds(0, 16)] = (x0 * f0 + (c0 * f20) * ev) * cv
                    msgb[e, pl.ds(16, 16)] = (x1 * f1 + (c1 * f21) * ev) * cv
                else:
                    msgb[e, pl.ds(0, 16)] = (x0 * f0) * cv
                    msgb[e, pl.ds(16, 16)] = (x1 * f1) * cv

        def clamp(k):
            if isinstance(k, int):
                return min(k, last)
            return jnp.minimum(k, last)

        def iter_body(k, P, Q):
            scatter_copy(Q).wait()
            dst_copy(clamp(k + 1), Q).start()
            src_copy(clamp(k + 1), Q).wait()
            issue_data(clamp(k + 1), Q)
            drain_data(k, P)
            src_copy(clamp(k + 2), P).start()
            compute(P)
            dst_copy(k, P).wait()
            scatter_start(P)

        pltpu.sync_copy(src_ref.at[pl.ds(e0_of(0), _CH)], src_a)
        pltpu.sync_copy(dst_ref.at[pl.ds(e0_of(0), _CH)], dst_a)
        issue_data(0, set_a)
        src_copy(1, set_b).start()
        dst_copy(1, set_b).start()
        src_copy(1, set_b).wait()
        issue_data(1, set_b)
        drain_data(0, set_a)
        src_copy(2, set_a).start()
        compute(set_a)
        scatter_start(set_a)

        iter_body(1, set_b, set_a)

        def pair(kk, carry):
            k = 2 * kk
            iter_body(k, set_a, set_b)
            iter_body(k + 1, set_b, set_a)
            return carry
        lax.fori_loop(1, chunks // 2, pair, 0)

        scatter_copy(set_b).wait()
        drain_data(clamp(last + 1), set_a)
        src_copy(clamp(last + 2), set_b).wait()
        dst_copy(clamp(last + 1), set_a).wait()

    def run_scalar(table):
        run_group(table, False, 0)

    def run_vec(table, dd):
        run_group(table, True, dd)

    def flush(out_ref):
        each_rowchunk(
            lambda j: pltpu.sync_copy(acc.at[pl.ds(j * _ZROWS, _ZROWS)],
                                      out_ref.at[pl.ds(j * _ZROWS, _ZROWS)]))

    for gi in range(2):
        zero_acc()
        plsc.subcore_barrier()

        @pl.when(c == 0)
        def _():
            if gi == 0:
                run_scalar(ta)
            else:
                run_vec(t1, 1)

        @pl.when(c == 1)
        def _():
            if gi == 0:
                run_vec(t0, 0)
            else:
                run_vec(t2, 2)

        plsc.subcore_barrier()

        @pl.when(c == 0)
        def _():
            flush(m_s if gi == 0 else m_v1)

        @pl.when(c == 1)
        def _():
            flush(m_v0 if gi == 0 else m_v2)

        if gi == 0:
            plsc.subcore_barrier()


def _final_body(ms_ref, m0_ref, m1_ref, m2_ref, rs_ref, rv_ref,
                gw1_ref, gb1_ref, gw2_ref, gb2_ref, os_ref, ov_ref):
    ms = ms_ref[...]
    g = _silu(jnp.dot(ms, gw1_ref[...],
                      preferred_element_type=jnp.float32) + gb1_ref[...])
    g = jax.nn.sigmoid(jnp.dot(g, gw2_ref[...],
                               preferred_element_type=jnp.float32) + gb2_ref[...])
    ug_s = g[:, :_NS]
    ug_v = g[:, _NS:]
    os_ref[...] = ug_s * rs_ref[...] + (1.0 - ug_s) * ms
    mv = jnp.stack([m0_ref[...], m1_ref[...], m2_ref[...]], axis=1)
    ov_ref[...] = ug_v[:, None, :] * rv_ref[...] + (1.0 - ug_v[:, None, :]) * mv


def kernel(sender_node_state_scalar, sender_node_state_vector,
           receiver_node_state_scalar, receiver_node_state_vector,
           edge_state, edge_vector, edge_distance, edges,
           filter_W, filter_b, msg_W1, msg_b1, msg_W2, msg_b2,
           gate_W1, gate_b1, gate_W2, gate_b2):
    N, ns = sender_node_state_scalar.shape
    E = edges.shape[0]
    es_dim = edge_state.shape[1]
    assert ns == _NS
    assert N % _ZROWS == 0

    blk_e = _NTILE * _CH
    e_pad = ((E + blk_e - 1) // blk_e) * blk_e
    grid_e = e_pad // blk_e

    src = edges[:, 0].astype(jnp.int32)
    dst = edges[:, 1].astype(jnp.int32)
    pad = e_pad - E
    if pad:
        zpad = jnp.zeros((pad,), jnp.int32)
        src = jnp.concatenate([src, zpad])
        dst = jnp.concatenate([dst, zpad])

    f32 = jnp.float32

    Rn = 2000
    grid_n = N // Rn
    ta, t0, t1, t2 = pl.pallas_call(
        _node_body,
        grid=(grid_n,),
        in_specs=[
            pl.BlockSpec((Rn, ns), lambda i: (i, 0)),
            pl.BlockSpec((Rn, 3, ns), lambda i: (i, 0, 0)),
            pl.BlockSpec((ns, ns), lambda i: (0, 0)),
            pl.BlockSpec((1, ns), lambda i: (0, 0)),
            pl.BlockSpec((ns, 3 * ns), lambda i: (0, 0)),
            pl.BlockSpec((1, 3 * ns), lambda i: (0, 0)),
        ],
        out_specs=[
            pl.BlockSpec((Rn, 2 * ns), lambda i: (i, 0)),
            pl.BlockSpec((Rn, 2 * ns), lambda i: (i, 0)),
            pl.BlockSpec((Rn, 2 * ns), lambda i: (i, 0)),
            pl.BlockSpec((Rn, 2 * ns), lambda i: (i, 0)),
        ],
        out_shape=[
            jax.ShapeDtypeStruct((N, 2 * ns), f32),
            jax.ShapeDtypeStruct((N, 2 * ns), f32),
            jax.ShapeDtypeStruct((N, 2 * ns), f32),
            jax.ShapeDtypeStruct((N, 2 * ns), f32),
        ],
    )(sender_node_state_scalar, sender_node_state_vector,
      msg_W1, msg_b1.reshape(1, ns), msg_W2, msg_b2.reshape(1, 3 * ns))

    ed_a, aux_a = pl.pallas_call(
        functools.partial(_edge_body, e_total=E, blk=blk_e),
        grid=(grid_e,),
        in_specs=[
            pl.BlockSpec((es_dim, blk_e), lambda i: (0, i)),
            pl.BlockSpec((1, blk_e), lambda i: (0, i)),
            pl.BlockSpec((3, blk_e), lambda i: (0, i)),
            pl.BlockSpec((3 * ns, es_dim), lambda i: (0, 0)),
            pl.BlockSpec((3 * ns, 1), lambda i: (0, 0)),
        ],
        out_specs=[
            pl.BlockSpec((blk_e, 4 * ns), lambda i: (i, 0)),
            pl.BlockSpec((8, blk_e), lambda i: (0, i)),
        ],
        out_shape=[
            jax.ShapeDtypeStruct((e_pad, 4 * ns), f32),
            jax.ShapeDtypeStruct((8, e_pad), f32),
        ],
    )(edge_state.T, edge_distance.T, edge_vector.T,
      filter_W.T, filter_b.reshape(3 * ns, 1))

    mesh = plsc.VectorSubcoreMesh(core_axis_name="c", subcore_axis_name="s",
                                  num_cores=2, num_subcores=_NTILE)
    m_s, m_v0, m_v1, m_v2 = pl.kernel(
        functools.partial(_sc_body, n_nodes=N, e_pad=e_pad),
        out_type=[jax.ShapeDtypeStruct((N, ns), f32)] * 4,
        mesh=mesh,
        compiler_params=pltpu.CompilerParams(needs_layout_passes=False,
                                             use_tc_tiling_on_sc=False),
        scratch_types=(
            [pltpu.VMEM((_ZROWS, ns), f32)]
            + [pltpu.VMEM((_CH,), jnp.int32)] * 4
            + [pltpu.VMEM((_CH, 2 * ns), f32)] * 4
            + [pltpu.VMEM((_CH,), f32)] * 4
            + [pltpu.VMEM((_CH, ns), f32)] * 2
            + [pltpu.SemaphoreType.DMA] * 8
            + [pltpu.VMEM_SHARED((N, ns), f32)]
        ),
    )(src, dst, ta, t0, t1, t2, ed_a, aux_a)

    new_s, new_v = pl.pallas_call(
        _final_body,
        grid=(grid_n,),
        in_specs=[
            pl.BlockSpec((Rn, ns), lambda i: (i, 0)),
            pl.BlockSpec((Rn, ns), lambda i: (i, 0)),
            pl.BlockSpec((Rn, ns), lambda i: (i, 0)),
            pl.BlockSpec((Rn, ns), lambda i: (i, 0)),
            pl.BlockSpec((Rn, ns), lambda i: (i, 0)),
            pl.BlockSpec((Rn, 3, ns), lambda i: (i, 0, 0)),
            pl.BlockSpec((ns, 2 * ns), lambda i: (0, 0)),
            pl.BlockSpec((1, 2 * ns), lambda i: (0, 0)),
            pl.BlockSpec((2 * ns, 2 * ns), lambda i: (0, 0)),
            pl.BlockSpec((1, 2 * ns), lambda i: (0, 0)),
        ],
        out_specs=[
            pl.BlockSpec((Rn, ns), lambda i: (i, 0)),
            pl.BlockSpec((Rn, 3, ns), lambda i: (i, 0, 0)),
        ],
        out_shape=[
            jax.ShapeDtypeStruct((N, ns), f32),
            jax.ShapeDtypeStruct((N, 3, ns), f32),
        ],
    )(m_s, m_v0, m_v1, m_v2,
      receiver_node_state_scalar, receiver_node_state_vector,
      gate_W1, gate_b1.reshape(1, 2 * ns), gate_W2, gate_b2.reshape(1, 2 * ns))

    return new_s, new_v
